# baseline (speedup 1.0000x reference)
import functools

import jax
import jax.numpy as jnp
from jax import lax
from jax.experimental import pallas as pl
from jax.experimental.pallas import tpu as pltpu
from jax.experimental.pallas import tpu_sc as plsc

B = 2
S = 2048
D = 1024
E = 8
F = 4096
K = 2
T = B * S

BLK = 256
NB = (K * T) // BLK + E
P = NB * BLK
FC = 2048
NFC = F // FC

NC = 2
NS = 16
NW = NC * NS
TW = T // NW
RW = 16
NSUB = TW // RW


def _router_body(x_ref, wr_ref, br_ref, ix_ref, gv_ref, cnt_ref, aux_ref):
    x = x_ref[...]
    logits = jnp.dot(x, wr_ref[...], preferred_element_type=jnp.float32)
    logits = logits + br_ref[...]
    m = jnp.max(logits, axis=-1, keepdims=True)
    p = jnp.exp(logits - m)
    p = p / jnp.sum(p, axis=-1, keepdims=True)

    lane = lax.broadcasted_iota(jnp.int32, (T, E), 1)
    m1 = jnp.max(p, axis=-1, keepdims=True)
    i1 = jnp.min(jnp.where(p == m1, lane, E), axis=-1, keepdims=True)
    p2 = jnp.where(lane == i1, -jnp.inf, p)
    m2 = jnp.max(p2, axis=-1, keepdims=True)
    i2 = jnp.min(jnp.where(p2 == m2, lane, E), axis=-1, keepdims=True)

    den = m1 + m2
    g1 = m1 / den
    g2 = m2 / den
    sel1 = (lane == i1).astype(jnp.float32)
    sel2 = (lane == i2).astype(jnp.float32)

    ix_ref[...] = jnp.where(lane == 0, i1, jnp.where(lane == 1, i2, 0))
    gv_ref[...] = jnp.where(lane == 0, g1, 0.0) + jnp.where(lane == 1, g2, 0.0)

    cnt = jnp.sum(sel1 + sel2, axis=0, keepdims=True)
    cnt_ref[...] = cnt
    psum = jnp.sum(p, axis=0, keepdims=True)
    aux = (E / (T * K * T)) * jnp.sum(cnt * psum)
    aux_ref[...] = jnp.full((1, 1), aux, jnp.float32)


def _router(x2, W_router, b_router):
    return pl.pallas_call(
        _router_body,
        out_shape=(
            jax.ShapeDtypeStruct((T, E), jnp.int32),
            jax.ShapeDtypeStruct((T, E), jnp.float32),
            jax.ShapeDtypeStruct((1, E), jnp.float32),
            jax.ShapeDtypeStruct((1, 1), jnp.float32),
        ),
    )(x2, W_router, b_router.reshape(1, E))


AR = (K * T) // 128


def _pos_body(a_ref, cnt_ref, pos_ref, bexp_ref):
    a = a_ref[...]
    counts = cnt_ref[...]
    lane8 = lax.broadcasted_iota(jnp.int32, (1, E), 1)

    po = jnp.ceil(counts / BLK) * BLK
    upper8 = (lax.broadcasted_iota(jnp.int32, (E, E), 0)
              < lax.broadcasted_iota(jnp.int32, (E, E), 1)).astype(jnp.float32)
    offs = jnp.dot(po, upper8, preferred_element_type=jnp.float32)

    upper128 = (lax.broadcasted_iota(jnp.int32, (128, 128), 0)
                < lax.broadcasted_iota(jnp.int32, (128, 128), 1)).astype(jnp.float32)
    lower64 = (lax.broadcasted_iota(jnp.int32, (AR, AR), 1)
               < lax.broadcasted_iota(jnp.int32, (AR, AR), 0)).astype(jnp.float32)

    pos_acc = jnp.zeros((AR, 128), jnp.float32)
    for e in range(E):
        x = (a == e).astype(jnp.float32)
        within = jnp.dot(x, upper128, preferred_element_type=jnp.float32)
        rowtot = jnp.sum(x, axis=1, keepdims=True)
        rowoff = jnp.dot(lower64, rowtot, preferred_element_type=jnp.float32)
        offe = jnp.sum(jnp.where(lane8 == e, offs, 0.0))
        pos_acc = pos_acc + x * (offe + rowoff + within)
    pos_ref[...] = pos_acc.astype(jnp.int32)

    lane128 = lax.broadcasted_iota(jnp.int32, (1, 128), 1)
    bstart = (lane128 * BLK).astype(jnp.float32)
    be_acc = jnp.zeros((1, 128), jnp.float32)
    for e in range(E):
        offe = jnp.sum(jnp.where(lane8 == e, offs, 0.0))
        poe = jnp.sum(jnp.where(lane8 == e, po, 0.0))
        inside = (bstart >= offe) & (bstart < offe + poe)
        be_acc = be_acc + e * inside.astype(jnp.float32)
    bexp_ref[...] = be_acc.astype(jnp.int32)


def _positions(a, counts):
    return pl.pallas_call(
        _pos_body,
        out_shape=(
            jax.ShapeDtypeStruct((AR, 128), jnp.int32),
            jax.ShapeDtypeStruct((1, 128), jnp.int32),
        ),
    )(a, counts)


@functools.cache
def _sc_mesh():
    return plsc.VectorSubcoreMesh(
        core_axis_name="c", subcore_axis_name="s",
        num_cores=NC, num_subcores=NS)


def _dispatch_body(x2_hbm, pos_hbm, xs_hbm, rows_v, pos_v, sem):
    wid = lax.axis_index("s") * NC + lax.axis_index("c")
    pltpu.sync_copy(pos_hbm.at[wid], pos_v)
    for s in range(NSUB):
        tbase = wid * TW + s * RW
        pltpu.sync_copy(x2_hbm.at[pl.ds(tbase, RW)], rows_v)
        pltpu.async_copy(rows_v, xs_hbm.at[pos_v.at[0, s]], sem).wait()
        pltpu.async_copy(rows_v, xs_hbm.at[pos_v.at[1, s]], sem).wait()


def _dispatch(x2, pos_w):
    return pl.kernel(
        _dispatch_body,
        out_type=jax.ShapeDtypeStruct((P, D), jnp.float32),
        mesh=_sc_mesh(),
        scratch_types=[
            pltpu.VMEM((RW, D), jnp.float32),
            pltpu.VMEM((K, NSUB, RW), jnp.int32),
            pltpu.SemaphoreType.DMA,
        ],
    )(x2, pos_w)


def _ffn_body(be_ref, xs_ref, w1_ref, b1_ref, w2_ref, b2_ref, ys_ref):
    c = pl.program_id(0)
    x = xs_ref[...].astype(jnp.bfloat16)
    w1 = w1_ref[0].astype(jnp.bfloat16)
    h = jnp.dot(x, w1, preferred_element_type=jnp.float32) + b1_ref[0]
    h = jnp.maximum(h, 0.0).astype(jnp.bfloat16)
    w2 = w2_ref[0].astype(jnp.bfloat16)
    y = jnp.dot(h, w2, preferred_element_type=jnp.float32)
    ys_ref[...] = (y + jnp.where(c == 0, 1.0, 0.0) * b2_ref[0])[None]


def _ffn(bexp, xs, W1, b1, W2, b2):
    grid_spec = pltpu.PrefetchScalarGridSpec(
        num_scalar_prefetch=1,
        grid=(NFC, NB),
        in_specs=[
            pl.BlockSpec((BLK, D), lambda c, b, be: (b, 0)),
            pl.BlockSpec((1, D, FC), lambda c, b, be: (be[b], 0, c)),
            pl.BlockSpec((1, 1, FC), lambda c, b, be: (be[b], 0, c)),
            pl.BlockSpec((1, FC, D), lambda c, b, be: (be[b], c, 0)),
            pl.BlockSpec((1, 1, D), lambda c, b, be: (be[b], 0, 0)),
        ],
        out_specs=pl.BlockSpec((1, BLK, D), lambda c, b, be: (c, b, 0)),
    )
    return pl.pallas_call(
        _ffn_body,
        grid_spec=grid_spec,
        out_shape=jax.ShapeDtypeStruct((NFC, P, D), jnp.float32),
        compiler_params=pltpu.CompilerParams(
            dimension_semantics=("arbitrary", "arbitrary"),
        ),
    )(bexp, xs, W1, b1.reshape(E, 1, F), W2, b2.reshape(E, 1, D))


def _combine_body(ys_hbm, pos_hbm, poshi_hbm, gvb_hbm, out_hbm, a0_v, a1_v,
                  b0_v, b1_v, g0_v, g1_v, pos_v, poshi_v, sem):
    wid = lax.axis_index("s") * NC + lax.axis_index("c")
    pltpu.sync_copy(pos_hbm.at[wid], pos_v)
    pltpu.sync_copy(poshi_hbm.at[wid], poshi_v)
    for s in range(NSUB):
        tbase = wid * TW + s * RW
        cps = [
            pltpu.async_copy(ys_hbm.at[pos_v.at[0, s]], a0_v, sem),
            pltpu.async_copy(ys_hbm.at[poshi_v.at[0, s]], a1_v, sem),
            pltpu.async_copy(ys_hbm.at[pos_v.at[1, s]], b0_v, sem),
            pltpu.async_copy(ys_hbm.at[poshi_v.at[1, s]], b1_v, sem),
        ]
        pltpu.sync_copy(gvb_hbm.at[0, pl.ds(tbase, RW)], g0_v)
        pltpu.sync_copy(gvb_hbm.at[1, pl.ds(tbase, RW)], g1_v)
        for cp in cps:
            cp.wait()

        def row(i, _):
            g0 = g0_v[i]
            g1 = g1_v[i]
            for v in range(D // 16):
                sl = pl.ds(v * 16, 16)
                a0_v[i, sl] = ((a0_v[i, sl] + a1_v[i, sl]) * g0
                               + (b0_v[i, sl] + b1_v[i, sl]) * g1)
            return 0

        lax.fori_loop(0, RW, row, 0)
        pltpu.sync_copy(a0_v, out_hbm.at[pl.ds(tbase, RW)])


def _combine(ys2, pos_w, pos_w_hi, gvb):
    return pl.kernel(
        _combine_body,
        out_type=jax.ShapeDtypeStruct((T, D), jnp.float32),
        mesh=_sc_mesh(),
        scratch_types=[
            pltpu.VMEM((RW, D), jnp.float32),
            pltpu.VMEM((RW, D), jnp.float32),
            pltpu.VMEM((RW, D), jnp.float32),
            pltpu.VMEM((RW, D), jnp.float32),
            pltpu.VMEM((RW, 16), jnp.float32),
            pltpu.VMEM((RW, 16), jnp.float32),
            pltpu.VMEM((K, NSUB, RW), jnp.int32),
            pltpu.VMEM((K, NSUB, RW), jnp.int32),
            pltpu.SemaphoreType.DMA,
        ],
    )(ys2, pos_w, pos_w_hi, gvb)


def kernel(x, W_router, b_router, W1, b1, W2, b2):
    x2 = x.reshape(T, D)
    ix, gv, counts, aux = _router(x2, W_router, b_router)

    a = jnp.concatenate(
        [ix[:, 0].reshape(T // 128, 128), ix[:, 1].reshape(T // 128, 128)],
        axis=0)
    pos, bexp = _positions(a, counts)

    pos_w = pos.reshape(K, NW, NSUB, RW).transpose(1, 0, 2, 3)
    bexp_arr = bexp[0, :NB]
    gvb = jnp.broadcast_to(
        jnp.stack([gv[:, 0], gv[:, 1]])[:, :, None], (K, T, 16))

    xs = _dispatch(x2, pos_w)
    ys = _ffn(bexp_arr, xs, W1, b1, W2, b2)
    out2 = _combine(ys.reshape(NFC * P, D), pos_w, pos_w + P, gvb)
    return out2.reshape(B, S, D), aux[0, 0]


# speedup vs baseline: 1.1056x; 1.1056x over previous
import functools

import jax
import jax.numpy as jnp
from jax import lax
from jax.experimental import pallas as pl
from jax.experimental.pallas import tpu as pltpu
from jax.experimental.pallas import tpu_sc as plsc

B = 2
S = 2048
D = 1024
E = 8
F = 4096
K = 2
T = B * S

BLK = 512
NB = (K * T) // BLK + E
P = NB * BLK
FC = 2048

NC = 2
NS = 16
NW = NC * NS
TW = T // NW
RW = 32
NSUB = TW // RW


def _router_body(x_ref, wr_ref, br_ref, ix_ref, gv_ref, cnt_ref, aux_ref):
    x = x_ref[...]
    logits = jnp.dot(x, wr_ref[...], preferred_element_type=jnp.float32)
    logits = logits + br_ref[...]
    m = jnp.max(logits, axis=-1, keepdims=True)
    p = jnp.exp(logits - m)
    p = p / jnp.sum(p, axis=-1, keepdims=True)

    lane = lax.broadcasted_iota(jnp.int32, (T, E), 1)
    m1 = jnp.max(p, axis=-1, keepdims=True)
    i1 = jnp.min(jnp.where(p == m1, lane, E), axis=-1, keepdims=True)
    p2 = jnp.where(lane == i1, -jnp.inf, p)
    m2 = jnp.max(p2, axis=-1, keepdims=True)
    i2 = jnp.min(jnp.where(p2 == m2, lane, E), axis=-1, keepdims=True)

    den = m1 + m2
    g1 = m1 / den
    g2 = m2 / den
    sel1 = (lane == i1).astype(jnp.float32)
    sel2 = (lane == i2).astype(jnp.float32)

    ix_ref[...] = jnp.where(lane == 0, i1, jnp.where(lane == 1, i2, 0))
    gv_ref[...] = jnp.where(lane == 0, g1, 0.0) + jnp.where(lane == 1, g2, 0.0)

    cnt = jnp.sum(sel1 + sel2, axis=0, keepdims=True)
    cnt_ref[...] = cnt
    psum = jnp.sum(p, axis=0, keepdims=True)
    aux = (E / (T * K * T)) * jnp.sum(cnt * psum)
    aux_ref[...] = jnp.full((1, 1), aux, jnp.float32)


def _router(x2, W_router, b_router):
    return pl.pallas_call(
        _router_body,
        out_shape=(
            jax.ShapeDtypeStruct((T, E), jnp.int32),
            jax.ShapeDtypeStruct((T, E), jnp.float32),
            jax.ShapeDtypeStruct((1, E), jnp.float32),
            jax.ShapeDtypeStruct((1, 1), jnp.float32),
        ),
    )(x2, W_router, b_router.reshape(1, E))


AR = (K * T) // 128


def _pos_body(a_ref, cnt_ref, pos_ref, bexp_ref):
    a = a_ref[...]
    counts = cnt_ref[...]
    lane8 = lax.broadcasted_iota(jnp.int32, (1, E), 1)

    po = jnp.ceil(counts / BLK) * BLK
    upper8 = (lax.broadcasted_iota(jnp.int32, (E, E), 0)
              < lax.broadcasted_iota(jnp.int32, (E, E), 1)).astype(jnp.float32)
    offs = jnp.dot(po, upper8, preferred_element_type=jnp.float32)

    upper128 = (lax.broadcasted_iota(jnp.int32, (128, 128), 0)
                < lax.broadcasted_iota(jnp.int32, (128, 128), 1)).astype(jnp.float32)
    lower64 = (lax.broadcasted_iota(jnp.int32, (AR, AR), 1)
               < lax.broadcasted_iota(jnp.int32, (AR, AR), 0)).astype(jnp.float32)

    pos_acc = jnp.zeros((AR, 128), jnp.float32)
    for e in range(E):
        x = (a == e).astype(jnp.float32)
        within = jnp.dot(x, upper128, preferred_element_type=jnp.float32)
        rowtot = jnp.sum(x, axis=1, keepdims=True)
        rowoff = jnp.dot(lower64, rowtot, preferred_element_type=jnp.float32)
        offe = jnp.sum(jnp.where(lane8 == e, offs, 0.0))
        pos_acc = pos_acc + x * (offe + rowoff + within)
    pos_ref[...] = pos_acc.astype(jnp.int32)

    lane128 = lax.broadcasted_iota(jnp.int32, (1, 128), 1)
    bstart = (lane128 * BLK).astype(jnp.float32)
    be_acc = jnp.zeros((1, 128), jnp.float32)
    for e in range(E):
        offe = jnp.sum(jnp.where(lane8 == e, offs, 0.0))
        poe = jnp.sum(jnp.where(lane8 == e, po, 0.0))
        inside = (bstart >= offe) & (bstart < offe + poe)
        be_acc = be_acc + e * inside.astype(jnp.float32)
    bexp_ref[...] = be_acc.astype(jnp.int32)


def _positions(a, counts):
    return pl.pallas_call(
        _pos_body,
        out_shape=(
            jax.ShapeDtypeStruct((AR, 128), jnp.int32),
            jax.ShapeDtypeStruct((1, 128), jnp.int32),
        ),
    )(a, counts)


@functools.cache
def _sc_mesh():
    return plsc.VectorSubcoreMesh(
        core_axis_name="c", subcore_axis_name="s",
        num_cores=NC, num_subcores=NS)


def _dispatch_body(x2_hbm, pos_hbm, xs_hbm, rows_v, pos_v, sem):
    wid = lax.axis_index("s") * NC + lax.axis_index("c")
    pltpu.sync_copy(pos_hbm.at[wid], pos_v)
    for s in range(NSUB):
        tbase = wid * TW + s * RW
        pltpu.sync_copy(x2_hbm.at[pl.ds(tbase, RW)], rows_v)
        pltpu.async_copy(rows_v, xs_hbm.at[pos_v.at[0, s]], sem).wait()
        pltpu.async_copy(rows_v, xs_hbm.at[pos_v.at[1, s]], sem).wait()


def _dispatch(x2, pos_w):
    return pl.kernel(
        _dispatch_body,
        out_type=jax.ShapeDtypeStruct((P, D), jnp.float32),
        mesh=_sc_mesh(),
        scratch_types=[
            pltpu.VMEM((RW, D), jnp.float32),
            pltpu.VMEM((K, NSUB, RW), jnp.int32),
            pltpu.SemaphoreType.DMA,
        ],
    )(x2, pos_w)


def _ffn_body(be_ref, xs_ref, w1_ref, b1_ref, w2_ref, b2_ref, ys_ref):
    c = pl.program_id(1)
    x = xs_ref[...]
    h = jnp.dot(x, w1_ref[0], preferred_element_type=jnp.float32) + b1_ref[0]
    h = jnp.maximum(h, 0.0)
    y = jnp.dot(h, w2_ref[0], preferred_element_type=jnp.float32)

    @pl.when(c == 0)
    def _():
        ys_ref[...] = y + b2_ref[0]

    @pl.when(c > 0)
    def _():
        ys_ref[...] += y


def _ffn(bexp, xs, W1, b1, W2, b2):
    grid_spec = pltpu.PrefetchScalarGridSpec(
        num_scalar_prefetch=1,
        grid=(NB, F // FC),
        in_specs=[
            pl.BlockSpec((BLK, D), lambda b, c, be: (b, 0)),
            pl.BlockSpec((1, D, FC), lambda b, c, be: (be[b], 0, c)),
            pl.BlockSpec((1, 1, FC), lambda b, c, be: (be[b], 0, c)),
            pl.BlockSpec((1, FC, D), lambda b, c, be: (be[b], c, 0)),
            pl.BlockSpec((1, 1, D), lambda b, c, be: (be[b], 0, 0)),
        ],
        out_specs=pl.BlockSpec((BLK, D), lambda b, c, be: (b, 0)),
    )
    return pl.pallas_call(
        _ffn_body,
        grid_spec=grid_spec,
        out_shape=jax.ShapeDtypeStruct((P, D), jnp.float32),
        compiler_params=pltpu.CompilerParams(
            dimension_semantics=("arbitrary", "arbitrary"),
        ),
    )(bexp, xs, W1, b1.reshape(E, 1, F), W2, b2.reshape(E, 1, D))


def _combine_body(ys_hbm, pos_hbm, gvb_hbm, out_hbm, a_v, b_v, g0_v, g1_v,
                  pos_v, sem):
    wid = lax.axis_index("s") * NC + lax.axis_index("c")
    pltpu.sync_copy(pos_hbm.at[wid], pos_v)
    for s in range(NSUB):
        tbase = wid * TW + s * RW
        pltpu.async_copy(ys_hbm.at[pos_v.at[0, s]], a_v, sem).wait()
        pltpu.async_copy(ys_hbm.at[pos_v.at[1, s]], b_v, sem).wait()
        pltpu.sync_copy(gvb_hbm.at[0, pl.ds(tbase, RW)], g0_v)
        pltpu.sync_copy(gvb_hbm.at[1, pl.ds(tbase, RW)], g1_v)

        def row(i, _):
            g0 = g0_v[i]
            g1 = g1_v[i]
            for v in range(D // 16):
                sl = pl.ds(v * 16, 16)
                a_v[i, sl] = a_v[i, sl] * g0 + b_v[i, sl] * g1
            return 0

        lax.fori_loop(0, RW, row, 0)
        pltpu.sync_copy(a_v, out_hbm.at[pl.ds(tbase, RW)])


def _combine(ys, pos_w, gvb):
    return pl.kernel(
        _combine_body,
        out_type=jax.ShapeDtypeStruct((T, D), jnp.float32),
        mesh=_sc_mesh(),
        scratch_types=[
            pltpu.VMEM((RW, D), jnp.float32),
            pltpu.VMEM((RW, D), jnp.float32),
            pltpu.VMEM((RW, 16), jnp.float32),
            pltpu.VMEM((RW, 16), jnp.float32),
            pltpu.VMEM((K, NSUB, RW), jnp.int32),
            pltpu.SemaphoreType.DMA,
        ],
    )(ys, pos_w, gvb)


def kernel(x, W_router, b_router, W1, b1, W2, b2):
    x2 = x.reshape(T, D)
    ix, gv, counts, aux = _router(x2, W_router, b_router)

    a = jnp.concatenate(
        [ix[:, 0].reshape(T // 128, 128), ix[:, 1].reshape(T // 128, 128)],
        axis=0)
    pos, bexp = _positions(a, counts)

    pos_w = pos.reshape(K, NW, NSUB, RW).transpose(1, 0, 2, 3)
    bexp_arr = bexp[0, :NB]
    gvb = jnp.broadcast_to(
        jnp.stack([gv[:, 0], gv[:, 1]])[:, :, None], (K, T, 16))

    xs = _dispatch(x2, pos_w)
    ys = _ffn(bexp_arr, xs, W1, b1, W2, b2)
    out2 = _combine(ys, pos_w, gvb)
    return out2.reshape(B, S, D), aux[0, 0]


# speedup vs baseline: 1.2926x; 1.1691x over previous
import functools

import jax
import jax.numpy as jnp
from jax import lax
from jax.experimental import pallas as pl
from jax.experimental.pallas import tpu as pltpu
from jax.experimental.pallas import tpu_sc as plsc

B = 2
S = 2048
D = 1024
E = 8
F = 4096
K = 2
T = B * S

BLK = 512
NB = (K * T) // BLK + E
P = NB * BLK
FC = 2048

NC = 2
NS = 16
NW = NC * NS
TW = T // NW
RW = 32
NSUB = TW // RW


def _router_body(x_ref, wr_ref, br_ref, ix_ref, gv_ref, cnt_ref, aux_ref):
    x = x_ref[...]
    logits = jnp.dot(x, wr_ref[...], preferred_element_type=jnp.float32)
    logits = logits + br_ref[...]
    m = jnp.max(logits, axis=-1, keepdims=True)
    p = jnp.exp(logits - m)
    p = p / jnp.sum(p, axis=-1, keepdims=True)

    lane = lax.broadcasted_iota(jnp.int32, (T, E), 1)
    m1 = jnp.max(p, axis=-1, keepdims=True)
    i1 = jnp.min(jnp.where(p == m1, lane, E), axis=-1, keepdims=True)
    p2 = jnp.where(lane == i1, -jnp.inf, p)
    m2 = jnp.max(p2, axis=-1, keepdims=True)
    i2 = jnp.min(jnp.where(p2 == m2, lane, E), axis=-1, keepdims=True)

    den = m1 + m2
    g1 = m1 / den
    g2 = m2 / den
    sel1 = (lane == i1).astype(jnp.float32)
    sel2 = (lane == i2).astype(jnp.float32)

    ix_ref[...] = jnp.where(lane == 0, i1, jnp.where(lane == 1, i2, 0))
    gv_ref[...] = jnp.where(lane == 0, g1, 0.0) + jnp.where(lane == 1, g2, 0.0)

    cnt = jnp.sum(sel1 + sel2, axis=0, keepdims=True)
    cnt_ref[...] = cnt
    psum = jnp.sum(p, axis=0, keepdims=True)
    aux = (E / (T * K * T)) * jnp.sum(cnt * psum)
    aux_ref[...] = jnp.full((1, 1), aux, jnp.float32)


def _router(x2, W_router, b_router):
    return pl.pallas_call(
        _router_body,
        out_shape=(
            jax.ShapeDtypeStruct((T, E), jnp.int32),
            jax.ShapeDtypeStruct((T, E), jnp.float32),
            jax.ShapeDtypeStruct((1, E), jnp.float32),
            jax.ShapeDtypeStruct((1, 1), jnp.float32),
        ),
    )(x2, W_router, b_router.reshape(1, E))


AR = (K * T) // 128


def _pos_body(a_ref, cnt_ref, pos_ref, bexp_ref):
    a = a_ref[...]
    counts = cnt_ref[...]
    lane8 = lax.broadcasted_iota(jnp.int32, (1, E), 1)

    po = jnp.ceil(counts / BLK) * BLK
    upper8 = (lax.broadcasted_iota(jnp.int32, (E, E), 0)
              < lax.broadcasted_iota(jnp.int32, (E, E), 1)).astype(jnp.float32)
    offs = jnp.dot(po, upper8, preferred_element_type=jnp.float32)

    upper128 = (lax.broadcasted_iota(jnp.int32, (128, 128), 0)
                < lax.broadcasted_iota(jnp.int32, (128, 128), 1)).astype(jnp.float32)
    lower64 = (lax.broadcasted_iota(jnp.int32, (AR, AR), 1)
               < lax.broadcasted_iota(jnp.int32, (AR, AR), 0)).astype(jnp.float32)

    pos_acc = jnp.zeros((AR, 128), jnp.float32)
    for e in range(E):
        x = (a == e).astype(jnp.float32)
        within = jnp.dot(x, upper128, preferred_element_type=jnp.float32)
        rowtot = jnp.sum(x, axis=1, keepdims=True)
        rowoff = jnp.dot(lower64, rowtot, preferred_element_type=jnp.float32)
        offe = jnp.sum(jnp.where(lane8 == e, offs, 0.0))
        pos_acc = pos_acc + x * (offe + rowoff + within)
    pos_ref[...] = pos_acc.astype(jnp.int32)

    lane128 = lax.broadcasted_iota(jnp.int32, (1, 128), 1)
    bstart = (lane128 * BLK).astype(jnp.float32)
    be_acc = jnp.zeros((1, 128), jnp.float32)
    for e in range(E):
        offe = jnp.sum(jnp.where(lane8 == e, offs, 0.0))
        poe = jnp.sum(jnp.where(lane8 == e, po, 0.0))
        inside = (bstart >= offe) & (bstart < offe + poe)
        be_acc = be_acc + e * inside.astype(jnp.float32)
    bexp_ref[...] = be_acc.astype(jnp.int32)


def _positions(a, counts):
    return pl.pallas_call(
        _pos_body,
        out_shape=(
            jax.ShapeDtypeStruct((AR, 128), jnp.int32),
            jax.ShapeDtypeStruct((1, 128), jnp.int32),
        ),
    )(a, counts)


@functools.cache
def _sc_mesh():
    return plsc.VectorSubcoreMesh(
        core_axis_name="c", subcore_axis_name="s",
        num_cores=NC, num_subcores=NS)


def _dispatch_body(x2_hbm, pos_hbm, xs_hbm, rows_v, pos_v, sem):
    wid = lax.axis_index("s") * NC + lax.axis_index("c")
    pltpu.sync_copy(pos_hbm.at[wid], pos_v)
    for s in range(NSUB):
        tbase = wid * TW + s * RW
        pltpu.sync_copy(x2_hbm.at[pl.ds(tbase, RW)], rows_v)
        pltpu.async_copy(rows_v, xs_hbm.at[pos_v.at[0, s]], sem).wait()
        pltpu.async_copy(rows_v, xs_hbm.at[pos_v.at[1, s]], sem).wait()


def _dispatch(x2, pos_w):
    return pl.kernel(
        _dispatch_body,
        out_type=jax.ShapeDtypeStruct((P, D), jnp.float32),
        mesh=_sc_mesh(),
        scratch_types=[
            pltpu.VMEM((RW, D), jnp.float32),
            pltpu.VMEM((K, NSUB, RW), jnp.int32),
            pltpu.SemaphoreType.DMA,
        ],
    )(x2, pos_w)


def _ffn_body(be_ref, xs_ref, w1_ref, b1_ref, w2_ref, b2_ref, ys_ref):
    c = pl.program_id(1)
    x = xs_ref[...]
    h = jnp.dot(x, w1_ref[0], preferred_element_type=jnp.float32) + b1_ref[0]
    h = jnp.maximum(h, 0.0)
    y = jnp.dot(h, w2_ref[0], preferred_element_type=jnp.float32)

    @pl.when(c == 0)
    def _():
        ys_ref[...] = y + b2_ref[0]

    @pl.when(c > 0)
    def _():
        ys_ref[...] += y


def _ffn(nb, bexp, xs, W1, b1, W2, b2):
    grid_spec = pltpu.PrefetchScalarGridSpec(
        num_scalar_prefetch=1,
        grid=(nb, F // FC),
        in_specs=[
            pl.BlockSpec((BLK, D), lambda b, c, be: (b, 0)),
            pl.BlockSpec((1, D, FC), lambda b, c, be: (be[b], 0, c)),
            pl.BlockSpec((1, 1, FC), lambda b, c, be: (be[b], 0, c)),
            pl.BlockSpec((1, FC, D), lambda b, c, be: (be[b], c, 0)),
            pl.BlockSpec((1, 1, D), lambda b, c, be: (be[b], 0, 0)),
        ],
        out_specs=pl.BlockSpec((BLK, D), lambda b, c, be: (b, 0)),
    )
    return pl.pallas_call(
        _ffn_body,
        grid_spec=grid_spec,
        out_shape=jax.ShapeDtypeStruct((P, D), jnp.float32),
        compiler_params=pltpu.CompilerParams(
            dimension_semantics=("arbitrary", "arbitrary"),
        ),
    )(bexp, xs, W1, b1.reshape(E, 1, F), W2, b2.reshape(E, 1, D))


def _combine_body(ys_hbm, pos_hbm, gvb_hbm, out_hbm, a_v, b_v, g0_v, g1_v,
                  pos_v, sem):
    wid = lax.axis_index("s") * NC + lax.axis_index("c")
    pltpu.sync_copy(pos_hbm.at[wid], pos_v)
    for s in range(NSUB):
        tbase = wid * TW + s * RW
        pltpu.async_copy(ys_hbm.at[pos_v.at[0, s]], a_v, sem).wait()
        pltpu.async_copy(ys_hbm.at[pos_v.at[1, s]], b_v, sem).wait()
        pltpu.sync_copy(gvb_hbm.at[0, pl.ds(tbase, RW)], g0_v)
        pltpu.sync_copy(gvb_hbm.at[1, pl.ds(tbase, RW)], g1_v)

        def row(i, _):
            g0 = g0_v[i]
            g1 = g1_v[i]
            for v in range(D // 16):
                sl = pl.ds(v * 16, 16)
                a_v[i, sl] = a_v[i, sl] * g0 + b_v[i, sl] * g1
            return 0

        lax.fori_loop(0, RW, row, 0)
        pltpu.sync_copy(a_v, out_hbm.at[pl.ds(tbase, RW)])


def _combine(ys, pos_w, gvb):
    return pl.kernel(
        _combine_body,
        out_type=jax.ShapeDtypeStruct((T, D), jnp.float32),
        mesh=_sc_mesh(),
        scratch_types=[
            pltpu.VMEM((RW, D), jnp.float32),
            pltpu.VMEM((RW, D), jnp.float32),
            pltpu.VMEM((RW, 16), jnp.float32),
            pltpu.VMEM((RW, 16), jnp.float32),
            pltpu.VMEM((K, NSUB, RW), jnp.int32),
            pltpu.SemaphoreType.DMA,
        ],
    )(ys, pos_w, gvb)


def kernel(x, W_router, b_router, W1, b1, W2, b2):
    x2 = x.reshape(T, D)
    ix, gv, counts, aux = _router(x2, W_router, b_router)

    a = jnp.concatenate(
        [ix[:, 0].reshape(T // 128, 128), ix[:, 1].reshape(T // 128, 128)],
        axis=0)
    pos, bexp = _positions(a, counts)

    pos_w = pos.reshape(K, NW, NSUB, RW).transpose(1, 0, 2, 3)
    bexp_arr = bexp[0, :NB]
    gvb = jnp.broadcast_to(
        jnp.stack([gv[:, 0], gv[:, 1]])[:, :, None], (K, T, 16))

    nb = jnp.sum(jnp.ceil(counts[0] / BLK)).astype(jnp.int32)
    xs = _dispatch(x2, pos_w)
    ys = _ffn(nb, bexp_arr, xs, W1, b1, W2, b2)
    out2 = _combine(ys, pos_w, gvb)
    return out2.reshape(B, S, D), aux[0, 0]
